# baseline (speedup 1.0000x reference)
import functools

import jax
import jax.numpy as jnp
from jax import lax
from jax.experimental import pallas as pl
from jax.experimental.pallas import tpu as pltpu
from jax.experimental.pallas import tpu_sc as plsc

PAD_IDX = 0
NEG_INF = -1e9

NC = 2
NS = 16
NW = NC * NS

GROUPS_PER_STEP = 8
CHUNK = GROUPS_PER_STEP * 128


def _sc_gather_kernel(n_rows, n_steps, bias_groups, D,
                      table, fidx, bidx, uidx, iidx, ubias, ibias,
                      rows_f, rows_b, ub_out, ib_out,
                      idx_v, rows_v0, rows_v1, bias_v, sem0, sem1):
    wid = lax.axis_index("s") * NC + lax.axis_index("c")
    rows_per_w = n_rows // NW
    idxrows_per_w = rows_per_w // 128
    bufs = (rows_v0, rows_v1)
    sems = (sem0, sem1)

    def fire(k, p):
        for j in range(GROUPS_PER_STEP):
            pltpu.async_copy(table.at[idx_v.at[k * GROUPS_PER_STEP + j]],
                             bufs[p].at[pl.ds(j * 128, 128)], sems[p])

    def drain(p):
        for j in range(GROUPS_PER_STEP):
            pltpu.make_async_copy(table.at[idx_v.at[j]],
                                  bufs[p].at[pl.ds(j * 128, 128)],
                                  sems[p]).wait()

    for idx_hbm, out_hbm in ((fidx, rows_f), (bidx, rows_b)):
        pltpu.sync_copy(
            idx_hbm.at[pl.ds(wid * idxrows_per_w, idxrows_per_w)], idx_v)
        fire(0, 0)

        def step2(m, carry, out_hbm=out_hbm):
            for j in range(2):
                k = 2 * m + j
                drain(j)
                fire(k + 1, 1 - j)
                pltpu.sync_copy(
                    bufs[j],
                    out_hbm.at[pl.ds(wid * rows_per_w + k * CHUNK, CHUNK)])
            return carry

        lax.fori_loop(0, (n_steps - 1) // 2, step2, 0)
        drain((n_steps - 1) % 2)
        pltpu.sync_copy(
            bufs[(n_steps - 1) % 2],
            out_hbm.at[pl.ds(wid * rows_per_w + (n_steps - 1) * CHUNK,
                             CHUNK)])

    half = NW // 2
    for active, bidx_hbm, btab, bout in ((wid < half, uidx, ubias, ub_out),
                                         (wid >= half, iidx, ibias, ib_out)):
        @pl.when(active)
        def _(bidx_hbm=bidx_hbm, btab=btab, bout=bout):
            lane = lax.rem(wid, half)
            pltpu.sync_copy(
                bidx_hbm.at[pl.ds(lane * bias_groups, bias_groups)],
                idx_v.at[pl.ds(0, bias_groups)])
            handles = [
                pltpu.async_copy(btab.at[idx_v.at[j]],
                                 bias_v.at[pl.ds(j * 128, 128)], sem0)
                for j in range(bias_groups)
            ]
            for h in handles:
                h.wait()
            pltpu.sync_copy(bias_v,
                            bout.at[pl.ds(lane * bias_groups * 128,
                                          bias_groups * 128)])


def _sc_gather(table, fidx, bidx, uidx, iidx, ubias, ibias, n_rows, B, D):
    n_steps = (n_rows // NW) // CHUNK
    bias_groups = (B // (NW // 2)) // 128
    mesh = plsc.VectorSubcoreMesh(core_axis_name="c", subcore_axis_name="s")
    body = functools.partial(_sc_gather_kernel, n_rows, n_steps, bias_groups, D)
    f = pl.kernel(
        body,
        out_type=(
            jax.ShapeDtypeStruct((n_rows, D), jnp.float32),
            jax.ShapeDtypeStruct((n_rows, D), jnp.float32),
            jax.ShapeDtypeStruct((B,), jnp.float32),
            jax.ShapeDtypeStruct((B,), jnp.float32),
        ),
        mesh=mesh,
        compiler_params=pltpu.CompilerParams(use_tc_tiling_on_sc=False),
        scratch_types=[
            pltpu.VMEM(((n_rows // NW) // 128, 128), jnp.int32),
            pltpu.VMEM((CHUNK, D), jnp.float32),
            pltpu.VMEM((CHUNK, D), jnp.float32),
            pltpu.VMEM((bias_groups * 128,), jnp.float32),
            pltpu.SemaphoreType.DMA,
            pltpu.SemaphoreType.DMA,
        ],
        name="sc_gather_rows_and_biases",
    )
    return f(table, fidx, bidx, uidx, iidx, ubias, ibias)


def _dot(a, b):
    return jnp.dot(a, b, precision=jax.lax.Precision.DEFAULT,
                   preferred_element_type=jnp.float32)


def _tc_pool_kernel(nblk, rf_hbm, rb_hbm, mf_ref, mb_ref, wm_ref, e_ref,
                    r_ref, ab_ref, ub_ref, ib_ref, gb_ref, o_ref,
                    fbuf0, fbuf1, bbuf0, bbuf1, dsem0, dsem1):
    absum = jnp.sum(ab_ref[...])
    BE = fbuf0.shape[0]
    i = pl.program_id(0)
    fbufs = (fbuf0, fbuf1)
    bbufs = (bbuf0, bbuf1)
    sems = (dsem0, dsem1)

    def fire(blk, p):
        pltpu.async_copy(rf_hbm.at[pl.ds(blk * BE, BE)], fbufs[p], sems[p])
        pltpu.async_copy(rb_hbm.at[pl.ds(blk * BE, BE)], bbufs[p], sems[p])

    def drain(p):
        pltpu.make_async_copy(rf_hbm.at[pl.ds(0, BE)], fbufs[p],
                              sems[p]).wait()
        pltpu.make_async_copy(rb_hbm.at[pl.ds(0, BE)], bbufs[p],
                              sems[p]).wait()

    slot = lax.rem(i, 2)

    @pl.when(i == 0)
    def _():
        fire(i, 0)

    @pl.when((i < nblk - 1) & (slot == 0))
    def _():
        fire(i + 1, 1)

    @pl.when((i < nblk - 1) & (slot == 1))
    def _():
        fire(i + 1, 0)

    @pl.when(slot == 0)
    def _():
        drain(0)

    @pl.when(slot == 1)
    def _():
        drain(1)

    def finish(rf_ref, rb_ref):
        _tc_pool_compute(absum, rf_ref, rb_ref, mf_ref, mb_ref, wm_ref,
                         e_ref, r_ref, ub_ref, ib_ref, gb_ref, o_ref)

    @pl.when(slot == 0)
    def _():
        finish(fbuf0, bbuf0)

    @pl.when(slot == 1)
    def _():
        finish(fbuf1, bbuf1)


def _tc_pool_compute(absum, rf_ref, rb_ref, mf_ref, mb_ref, wm_ref, e_ref,
                     r_ref, ub_ref, ib_ref, gb_ref, o_ref):
    def pool(rows, mask):
        s = _dot(rows, wm_ref[...]) + absum
        s = jnp.where(mask, s, NEG_INF)
        m = jnp.max(s, axis=-1, keepdims=True)
        e = jnp.exp(s - m)
        d = jnp.sum(e, axis=-1, keepdims=True)
        p = e / d
        pexp = _dot(p, e_ref[...])
        return _dot(pexp * rows, r_ref[...])

    pu = pool(rf_ref[...], mf_ref[...] != 0)
    pi = pool(rb_ref[...], mb_ref[...] != 0)
    dot = jnp.sum(pu * pi, axis=-1, keepdims=True)
    o_ref[0] = dot + ub_ref[0] + ib_ref[0] + gb_ref[0, 0]


def _tc_pool(rows_f, rows_b, fidx, bidx, ub, ib, w, ab, gb, B, L, D, BE=256):
    nblk = B // BE
    wmat = jnp.kron(jnp.eye(L, dtype=jnp.float32), w.reshape(D, 1))
    emat = jnp.kron(jnp.eye(L, dtype=jnp.float32),
                    jnp.ones((1, D), jnp.float32))
    rmat = jnp.kron(jnp.ones((L, 1), jnp.float32),
                    jnp.eye(D, dtype=jnp.float32))
    out = pl.pallas_call(
        functools.partial(_tc_pool_kernel, nblk),
        grid=(nblk,),
        in_specs=[
            pl.BlockSpec(memory_space=pl.ANY),
            pl.BlockSpec(memory_space=pl.ANY),
            pl.BlockSpec((BE, L), lambda i: (i, 0)),
            pl.BlockSpec((BE, L), lambda i: (i, 0)),
            pl.BlockSpec((L * D, L), lambda i: (0, 0)),
            pl.BlockSpec((L, L * D), lambda i: (0, 0)),
            pl.BlockSpec((L * D, D), lambda i: (0, 0)),
            pl.BlockSpec((1, D), lambda i: (0, 0)),
            pl.BlockSpec((1, BE, 1), lambda i: (i, 0, 0)),
            pl.BlockSpec((1, BE, 1), lambda i: (i, 0, 0)),
            pl.BlockSpec((1, 1), lambda i: (0, 0)),
        ],
        out_specs=pl.BlockSpec((1, BE, 1), lambda i: (i, 0, 0)),
        out_shape=jax.ShapeDtypeStruct((nblk, BE, 1), jnp.float32),
        scratch_shapes=[
            pltpu.VMEM((BE, L * D), jnp.float32),
            pltpu.VMEM((BE, L * D), jnp.float32),
            pltpu.VMEM((BE, L * D), jnp.float32),
            pltpu.VMEM((BE, L * D), jnp.float32),
            pltpu.SemaphoreType.DMA,
            pltpu.SemaphoreType.DMA,
        ],
    )(
        rows_f.reshape(B, L * D),
        rows_b.reshape(B, L * D),
        fidx, bidx,
        wmat, emat, rmat, ab.reshape(1, D),
        ub.reshape(nblk, BE, 1), ib.reshape(nblk, BE, 1),
        gb.reshape(1, 1),
    )
    return out.reshape(B)


def kernel(user_idx, item_idx, fav_subjects, book_subjects, subj_emb,
           attn_weight, attn_bias, user_bias, item_bias, global_bias):
    B, L = fav_subjects.shape
    D = subj_emb.shape[1]
    n_rows = B * L

    fidx = fav_subjects.astype(jnp.int32).reshape(n_rows // 128, 128)
    bidx = book_subjects.astype(jnp.int32).reshape(n_rows // 128, 128)
    uidx = user_idx.astype(jnp.int32).reshape(B // 128, 128)
    iidx = item_idx.astype(jnp.int32).reshape(B // 128, 128)

    rows_f, rows_b, ub, ib = _sc_gather(
        subj_emb, fidx, bidx, uidx, iidx,
        user_bias.reshape(-1), item_bias.reshape(-1), n_rows, B, D)

    return _tc_pool(rows_f, rows_b, fav_subjects.astype(jnp.int32),
                    book_subjects.astype(jnp.int32), ub, ib,
                    attn_weight, attn_bias, global_bias, B, L, D)


# speedup vs baseline: 1.0060x; 1.0060x over previous
import functools

import jax
import jax.numpy as jnp
from jax import lax
from jax.experimental import pallas as pl
from jax.experimental.pallas import tpu as pltpu
from jax.experimental.pallas import tpu_sc as plsc

PAD_IDX = 0
NEG_INF = -1e9

NC = 2
NS = 16
NW = NC * NS

GROUPS_PER_STEP = 8
CHUNK = GROUPS_PER_STEP * 128


def _sc_gather_kernel(n_rows, n_steps, bias_groups, D,
                      table, fidx, bidx, uidx, iidx, ubias, ibias,
                      rows_f, rows_b, ub_out, ib_out,
                      idx_v, rows_v0, rows_v1, bias_v, sem0, sem1):
    wid = lax.axis_index("s") * NC + lax.axis_index("c")
    rows_per_w = n_rows // NW
    idxrows_per_w = rows_per_w // 128
    bufs = (rows_v0, rows_v1)
    sems = (sem0, sem1)

    def fire(k, p):
        for j in range(GROUPS_PER_STEP):
            pltpu.async_copy(table.at[idx_v.at[k * GROUPS_PER_STEP + j]],
                             bufs[p].at[pl.ds(j * 128, 128)], sems[p])

    def drain(p):
        for j in range(GROUPS_PER_STEP):
            pltpu.make_async_copy(table.at[idx_v.at[j]],
                                  bufs[p].at[pl.ds(j * 128, 128)],
                                  sems[p]).wait()

    for idx_hbm, out_hbm in ((fidx, rows_f), (bidx, rows_b)):
        pltpu.sync_copy(
            idx_hbm.at[pl.ds(wid * idxrows_per_w, idxrows_per_w)], idx_v)
        fire(0, 0)

        def step2(m, carry, out_hbm=out_hbm):
            for j in range(2):
                k = 2 * m + j
                drain(j)
                fire(k + 1, 1 - j)
                pltpu.sync_copy(
                    bufs[j],
                    out_hbm.at[pl.ds(wid * rows_per_w + k * CHUNK, CHUNK)])
            return carry

        lax.fori_loop(0, (n_steps - 1) // 2, step2, 0)
        drain((n_steps - 1) % 2)
        pltpu.sync_copy(
            bufs[(n_steps - 1) % 2],
            out_hbm.at[pl.ds(wid * rows_per_w + (n_steps - 1) * CHUNK,
                             CHUNK)])

    half = NW // 2
    for active, bidx_hbm, btab, bout in ((wid < half, uidx, ubias, ub_out),
                                         (wid >= half, iidx, ibias, ib_out)):
        @pl.when(active)
        def _(bidx_hbm=bidx_hbm, btab=btab, bout=bout):
            lane = lax.rem(wid, half)
            pltpu.sync_copy(
                bidx_hbm.at[pl.ds(lane * bias_groups, bias_groups)],
                idx_v.at[pl.ds(0, bias_groups)])
            handles = [
                pltpu.async_copy(btab.at[idx_v.at[j]],
                                 bias_v.at[pl.ds(j * 128, 128)], sem0)
                for j in range(bias_groups)
            ]
            for h in handles:
                h.wait()
            pltpu.sync_copy(bias_v,
                            bout.at[pl.ds(lane * bias_groups * 128,
                                          bias_groups * 128)])


def _sc_gather(table, fidx, bidx, uidx, iidx, ubias, ibias, n_rows, B, D):
    n_steps = (n_rows // NW) // CHUNK
    bias_groups = (B // (NW // 2)) // 128
    mesh = plsc.VectorSubcoreMesh(core_axis_name="c", subcore_axis_name="s")
    body = functools.partial(_sc_gather_kernel, n_rows, n_steps, bias_groups, D)
    f = pl.kernel(
        body,
        out_type=(
            jax.ShapeDtypeStruct((n_rows, D), jnp.float32),
            jax.ShapeDtypeStruct((n_rows, D), jnp.float32),
            jax.ShapeDtypeStruct((B,), jnp.float32),
            jax.ShapeDtypeStruct((B,), jnp.float32),
        ),
        mesh=mesh,
        compiler_params=pltpu.CompilerParams(use_tc_tiling_on_sc=False),
        scratch_types=[
            pltpu.VMEM(((n_rows // NW) // 128, 128), jnp.int32),
            pltpu.VMEM((CHUNK, D), jnp.float32),
            pltpu.VMEM((CHUNK, D), jnp.float32),
            pltpu.VMEM((bias_groups * 128,), jnp.float32),
            pltpu.SemaphoreType.DMA,
            pltpu.SemaphoreType.DMA,
        ],
        name="sc_gather_rows_and_biases",
    )
    return f(table, fidx, bidx, uidx, iidx, ubias, ibias)


def _dot(a, b):
    return jnp.dot(a, b, precision=jax.lax.Precision.DEFAULT,
                   preferred_element_type=jnp.float32)


def _tc_pool_kernel(rf_ref, rb_ref, mf_ref, mb_ref, wm_ref, e_ref,
                    r_ref, ab_ref, ub_ref, ib_ref, gb_ref, o_ref):
    absum = jnp.sum(ab_ref[...])

    def pool(rows, mask):
        s = _dot(rows, wm_ref[...]) + absum
        s = jnp.where(mask, s, NEG_INF)
        m = jnp.max(s, axis=-1, keepdims=True)
        e = jnp.exp(s - m)
        d = jnp.sum(e, axis=-1, keepdims=True)
        p = e / d
        pexp = _dot(p, e_ref[...])
        return _dot(pexp * rows, r_ref[...])

    pu = pool(rf_ref[...], mf_ref[...] != 0)
    pi = pool(rb_ref[...], mb_ref[...] != 0)
    dot = jnp.sum(pu * pi, axis=-1, keepdims=True)
    o_ref[0] = dot + ub_ref[0] + ib_ref[0] + gb_ref[0, 0]


def _tc_pool(rows_f, rows_b, fidx, bidx, ub, ib, w, ab, gb, B, L, D, BE=256):
    nblk = B // BE
    wmat = jnp.kron(jnp.eye(L, dtype=jnp.float32), w.reshape(D, 1))
    emat = jnp.kron(jnp.eye(L, dtype=jnp.float32),
                    jnp.ones((1, D), jnp.float32))
    rmat = jnp.kron(jnp.ones((L, 1), jnp.float32),
                    jnp.eye(D, dtype=jnp.float32))
    out = pl.pallas_call(
        _tc_pool_kernel,
        grid=(nblk,),
        in_specs=[
            pl.BlockSpec((BE, L * D), lambda i: (i, 0)),
            pl.BlockSpec((BE, L * D), lambda i: (i, 0)),
            pl.BlockSpec((BE, L), lambda i: (i, 0)),
            pl.BlockSpec((BE, L), lambda i: (i, 0)),
            pl.BlockSpec((L * D, L), lambda i: (0, 0)),
            pl.BlockSpec((L, L * D), lambda i: (0, 0)),
            pl.BlockSpec((L * D, D), lambda i: (0, 0)),
            pl.BlockSpec((1, D), lambda i: (0, 0)),
            pl.BlockSpec((1, BE, 1), lambda i: (i, 0, 0)),
            pl.BlockSpec((1, BE, 1), lambda i: (i, 0, 0)),
            pl.BlockSpec((1, 1), lambda i: (0, 0)),
        ],
        out_specs=pl.BlockSpec((1, BE, 1), lambda i: (i, 0, 0)),
        out_shape=jax.ShapeDtypeStruct((nblk, BE, 1), jnp.float32),
    )(
        rows_f.reshape(B, L * D),
        rows_b.reshape(B, L * D),
        fidx, bidx,
        wmat, emat, rmat, ab.reshape(1, D),
        ub.reshape(nblk, BE, 1), ib.reshape(nblk, BE, 1),
        gb.reshape(1, 1),
    )
    return out.reshape(B)


def kernel(user_idx, item_idx, fav_subjects, book_subjects, subj_emb,
           attn_weight, attn_bias, user_bias, item_bias, global_bias):
    B, L = fav_subjects.shape
    D = subj_emb.shape[1]
    n_rows = B * L

    fidx = fav_subjects.astype(jnp.int32).reshape(n_rows // 128, 128)
    bidx = book_subjects.astype(jnp.int32).reshape(n_rows // 128, 128)
    uidx = user_idx.astype(jnp.int32).reshape(B // 128, 128)
    iidx = item_idx.astype(jnp.int32).reshape(B // 128, 128)

    rows_f, rows_b, ub, ib = _sc_gather(
        subj_emb, fidx, bidx, uidx, iidx,
        user_bias.reshape(-1), item_bias.reshape(-1), n_rows, B, D)

    return _tc_pool(rows_f, rows_b, fav_subjects.astype(jnp.int32),
                    book_subjects.astype(jnp.int32), ub, ib,
                    attn_weight, attn_bias, global_bias, B, L, D)
